# Initial kernel scaffold; baseline (speedup 1.0000x reference)
import jax
import jax.numpy as jnp
from jax.experimental import pallas as pl


def kernel(x, table):
    raise NotImplementedError("write your pallas kernel here")


# speedup vs baseline: 1.0936x; 1.0936x over previous
import functools

import jax
import jax.numpy as jnp
from jax import lax
from jax.experimental import pallas as pl
from jax.experimental.pallas import tpu as pltpu
from jax.experimental.pallas import tpu_sc as plsc

B, H, D = 16384, 50, 32
N = B * H
NW = 32
PER_W = N // NW
GW = 128
CB = 1024
K = CB // GW
NCHUNK = PER_W // CB
IDX_ROWS_W = PER_W // GW

_mesh = plsc.VectorSubcoreMesh(core_axis_name="c", subcore_axis_name="s")


@functools.partial(
    pl.kernel,
    mesh=_mesh,
    out_type=jax.ShapeDtypeStruct((N, D), jnp.float32),
    compiler_params=pltpu.CompilerParams(
        needs_layout_passes=False, use_tc_tiling_on_sc=False
    ),
    scratch_types=[
        pltpu.VMEM((K, GW), jnp.int32),
        pltpu.VMEM((CB, D), jnp.float32),
        pltpu.VMEM((16,), jnp.float32),
        pltpu.SemaphoreType.DMA,
    ],
)
def _emb(idx_hbm, table_hbm, out_hbm, idx_v, rows_v, mask_v, sem):
    wid = lax.axis_index("s") * 2 + lax.axis_index("c")

    def chunk_body(c, carry):
        base2d = wid * IDX_ROWS_W + c * K
        pltpu.sync_copy(idx_hbm.at[pl.ds(base2d, K)], idx_v)

        copies = []
        for j in range(K):
            copies.append(
                pltpu.async_copy(
                    table_hbm.at[idx_v.at[j]],
                    rows_v.at[pl.ds(j * GW, GW)],
                    sem,
                )
            )

        def det_body(j, acc):
            a = acc
            for t in range(GW // 16):
                v = idx_v[j, pl.ds(t * 16, 16)]
                a = a + jnp.where(v == 0, 1, 0).astype(jnp.int32)
            return a

        acc = lax.fori_loop(0, K, det_body, jnp.zeros((16,), jnp.int32))
        any_pad = jnp.sum(acc) > 0

        for cp in copies:
            cp.wait()

        @pl.when(any_pad)
        def _mask_pass():
            def grp_body(g, carry2):
                vi = idx_v[g // (GW // 16), pl.ds((g % (GW // 16)) * 16, 16)]
                mask_v[...] = jnp.where(vi == 0, 0.0, 1.0).astype(jnp.float32)
                for r in range(16):
                    em = plsc.load_gather(
                        mask_v, [jnp.full((16,), r, jnp.int32)]
                    )
                    row = g * 16 + r
                    for h2 in range(2):
                        cur = rows_v[row, pl.ds(h2 * 16, 16)]
                        rows_v[row, pl.ds(h2 * 16, 16)] = cur * em
                return carry2

            lax.fori_loop(0, CB // 16, grp_body, 0)

        pltpu.sync_copy(rows_v, out_hbm.at[pl.ds(wid * PER_W + c * CB, CB)])
        return carry

    lax.fori_loop(0, NCHUNK, chunk_body, 0)


def kernel(x, table):
    idx2d = x.astype(jnp.int32).reshape(N // GW, GW)
    out = _emb(idx2d, table)
    return out.reshape(B, H, D)


# speedup vs baseline: 1.1127x; 1.0174x over previous
import functools

import jax
import jax.numpy as jnp
from jax import lax
from jax.experimental import pallas as pl
from jax.experimental.pallas import tpu as pltpu
from jax.experimental.pallas import tpu_sc as plsc

B, H, D = 16384, 50, 32
N = B * H
NW = 32
PER_W = N // NW
GW = 128
CB = 1024
K = CB // GW
NCHUNK = PER_W // CB
IDX_ROWS_W = PER_W // GW

NPAIR = NCHUNK // 2

_mesh = plsc.VectorSubcoreMesh(core_axis_name="c", subcore_axis_name="s")


@functools.partial(
    pl.kernel,
    mesh=_mesh,
    out_type=jax.ShapeDtypeStruct((N, D), jnp.float32),
    compiler_params=pltpu.CompilerParams(
        needs_layout_passes=False, use_tc_tiling_on_sc=False
    ),
    scratch_types=[
        pltpu.VMEM((K, GW), jnp.int32),
        pltpu.VMEM((K, GW), jnp.int32),
        pltpu.VMEM((CB, D), jnp.float32),
        pltpu.VMEM((CB, D), jnp.float32),
        pltpu.VMEM((16,), jnp.float32),
        pltpu.SemaphoreType.DMA,
        pltpu.SemaphoreType.DMA,
        pltpu.SemaphoreType.DMA,
        pltpu.SemaphoreType.DMA,
    ],
)
def _emb(idx_hbm, table_hbm, out_hbm, idx_v0, idx_v1, rows_v0, rows_v1,
         mask_v, sem_g0, sem_g1, sem_o0, sem_o1):
    wid = lax.axis_index("s") * 2 + lax.axis_index("c")
    idx_v = (idx_v0, idx_v1)
    rows_v = (rows_v0, rows_v1)
    sem_g = (sem_g0, sem_g1)
    sem_o = (sem_o0, sem_o1)

    def fire(b, c):
        base2d = wid * IDX_ROWS_W + c * K
        pltpu.sync_copy(idx_hbm.at[pl.ds(base2d, K)], idx_v[b])
        for j in range(K):
            pltpu.async_copy(
                table_hbm.at[idx_v[b].at[j]],
                rows_v[b].at[pl.ds(j * GW, GW)],
                sem_g[b],
            )

    def drain_gathers(b):
        for j in range(K):
            pltpu.make_async_copy(
                table_hbm.at[idx_v[b].at[j]],
                rows_v[b].at[pl.ds(j * GW, GW)],
                sem_g[b],
            ).wait()

    def out_fire(b, c):
        pltpu.async_copy(
            rows_v[b], out_hbm.at[pl.ds(wid * PER_W + c * CB, CB)], sem_o[b]
        )

    def out_drain(b, c):
        pltpu.make_async_copy(
            rows_v[b], out_hbm.at[pl.ds(wid * PER_W + c * CB, CB)], sem_o[b]
        ).wait()

    def process(b, c):
        def det_body(j, acc):
            a = acc
            for t in range(GW // 16):
                v = idx_v[b][j, pl.ds(t * 16, 16)]
                a = a + jnp.where(v == 0, 1, 0).astype(jnp.int32)
            return a

        acc = lax.fori_loop(0, K, det_body, jnp.zeros((16,), jnp.int32))
        any_pad = jnp.sum(acc) > 0

        drain_gathers(b)

        @pl.when(any_pad)
        def _mask_pass():
            def grp_body(g, carry2):
                vi = idx_v[b][g // (GW // 16), pl.ds((g % (GW // 16)) * 16, 16)]
                mask_v[...] = jnp.where(vi == 0, 0.0, 1.0).astype(jnp.float32)
                for r in range(16):
                    em = plsc.load_gather(
                        mask_v, [jnp.full((16,), r, jnp.int32)]
                    )
                    row = g * 16 + r
                    for h2 in range(2):
                        cur = rows_v[b][row, pl.ds(h2 * 16, 16)]
                        rows_v[b][row, pl.ds(h2 * 16, 16)] = cur * em
                return carry2

            lax.fori_loop(0, CB // 16, grp_body, 0)

        out_fire(b, c)

    fire(0, 0)

    def pair_body(c2, carry):
        a = 2 * c2
        fire(1, a + 1)
        process(0, a)
        process(1, a + 1)
        out_drain(0, a)
        fire(0, a + 2)
        out_drain(1, a + 1)
        return carry

    lax.fori_loop(0, NPAIR, pair_body, 0)

    process(0, NCHUNK - 1)
    out_drain(0, NCHUNK - 1)


def kernel(x, table):
    idx2d = x.astype(jnp.int32).reshape(N // GW, GW)
    out = _emb(idx2d, table)
    return out.reshape(B, H, D)


# speedup vs baseline: 1.9241x; 1.7292x over previous
import functools

import jax
import jax.numpy as jnp
from jax import lax
from jax.experimental import pallas as pl
from jax.experimental.pallas import tpu as pltpu
from jax.experimental.pallas import tpu_sc as plsc

B, H, D = 16384, 50, 32
NW = 32
BW = B // NW
GW = 128
K = BW // GW
NPAIR = H // 2

_mesh = plsc.VectorSubcoreMesh(core_axis_name="c", subcore_axis_name="s")


@functools.partial(
    pl.kernel,
    mesh=_mesh,
    out_type=jax.ShapeDtypeStruct((H, B, D), jnp.float32),
    compiler_params=pltpu.CompilerParams(
        needs_layout_passes=False, use_tc_tiling_on_sc=False
    ),
    scratch_types=[
        pltpu.VMEM((K, GW), jnp.int32),
        pltpu.VMEM((K, GW), jnp.int32),
        pltpu.VMEM((BW, D), jnp.float32),
        pltpu.VMEM((BW, D), jnp.float32),
        pltpu.VMEM((16,), jnp.float32),
        pltpu.SemaphoreType.DMA,
        pltpu.SemaphoreType.DMA,
        pltpu.SemaphoreType.DMA,
        pltpu.SemaphoreType.DMA,
    ],
)
def _emb(idx_hbm, table_hbm, out_hbm, idx_v0, idx_v1, rows_v0, rows_v1,
         mask_v, sem_g0, sem_g1, sem_o0, sem_o1):
    wid = lax.axis_index("s") * 2 + lax.axis_index("c")
    idx_v = (idx_v0, idx_v1)
    rows_v = (rows_v0, rows_v1)
    sem_g = (sem_g0, sem_g1)
    sem_o = (sem_o0, sem_o1)

    def fire(b, h):
        pltpu.sync_copy(idx_hbm.at[h, pl.ds(wid * K, K)], idx_v[b])
        for j in range(K):
            pltpu.async_copy(
                table_hbm.at[idx_v[b].at[j]],
                rows_v[b].at[pl.ds(j * GW, GW)],
                sem_g[b],
            )

    def drain_gathers(b):
        for j in range(K):
            pltpu.make_async_copy(
                table_hbm.at[idx_v[b].at[j]],
                rows_v[b].at[pl.ds(j * GW, GW)],
                sem_g[b],
            ).wait()

    def out_fire(b, h):
        pltpu.async_copy(
            rows_v[b], out_hbm.at[h, pl.ds(wid * BW, BW)], sem_o[b]
        )

    def out_drain(b, h):
        pltpu.make_async_copy(
            rows_v[b], out_hbm.at[h, pl.ds(wid * BW, BW)], sem_o[b]
        ).wait()

    def process(b, h):
        def det_body(j, acc):
            a = acc
            for t in range(GW // 16):
                v = idx_v[b][j, pl.ds(t * 16, 16)]
                a = a + jnp.where(v == 0, 1, 0).astype(jnp.int32)
            return a

        acc = lax.fori_loop(0, K, det_body, jnp.zeros((16,), jnp.int32))
        any_pad = jnp.sum(acc) > 0

        drain_gathers(b)

        @pl.when(any_pad)
        def _mask_pass():
            def grp_body(g, carry2):
                vi = idx_v[b][g // (GW // 16), pl.ds((g % (GW // 16)) * 16, 16)]
                mask_v[...] = jnp.where(vi == 0, 0.0, 1.0).astype(jnp.float32)
                for r in range(16):
                    em = plsc.load_gather(
                        mask_v, [jnp.full((16,), r, jnp.int32)]
                    )
                    row = g * 16 + r
                    for h2 in range(2):
                        cur = rows_v[b][row, pl.ds(h2 * 16, 16)]
                        rows_v[b][row, pl.ds(h2 * 16, 16)] = cur * em
                return carry2

            lax.fori_loop(0, BW // 16, grp_body, 0)

        out_fire(b, h)

    fire(0, 0)

    def pair_body(c2, carry):
        a = 2 * c2
        fire(1, a + 1)
        process(0, a)
        process(1, a + 1)
        out_drain(0, a)

        @pl.when(c2 < NPAIR - 1)
        def _():
            fire(0, a + 2)

        out_drain(1, a + 1)
        return carry

    lax.fori_loop(0, NPAIR, pair_body, 0)


def kernel(x, table):
    idx3d = jnp.transpose(x.astype(jnp.int32)).reshape(H, B // GW, GW)
    out_t = _emb(idx3d, table)
    return jnp.transpose(out_t, (1, 0, 2))
